# Initial kernel scaffold; baseline (speedup 1.0000x reference)
import jax
import jax.numpy as jnp
from jax.experimental import pallas as pl


def kernel(q_in, kv_in, Wq, bq, Wk, bk, Wv, bv, Woff, boff, Waw, baw, Wout, bout):
    raise NotImplementedError("write your pallas kernel here")


# speedup vs baseline: 27.2946x; 27.2946x over previous
import functools
import math

import jax
import jax.numpy as jnp
from jax import lax
from jax.experimental import pallas as pl
from jax.experimental.pallas import tpu as pltpu
from jax.experimental.pallas import tpu_sc as plsc

DM = 1024
H = 16
P = 4
D = DM // H
HP = H * P
L_SEQ = 2048
B_SZ = 2

BLKA = 512
CH = 64
ROWS = CH * P * 2
NCHUNK = L_SEQ // CH

_NC = 2
_NS = 16


def _proj_body(xq, xkv, wq, bq, wk, bk, wv, bv, woff, boff, waw, baw,
               q_o, k_o, v_o, g0_o, g1_o, w1_o, lg_o):
    i = pl.program_id(0)
    x = xq[...]
    y = xkv[...]

    def mm(a, w):
        return lax.dot_general(a, w, (((1,), (1,)), ((), ())),
                               preferred_element_type=jnp.float32)

    q_o[...] = (mm(x, wq[...]) + bq[...]) * (1.0 / math.sqrt(D))
    k_o[...] = mm(y, wk[...]) + bk[...]
    v_o[...] = mm(y, wv[...]) + bv[...]
    off = mm(x, woff[...]) + boff[...]
    lg_o[...] = mm(x, waw[...]) + baw[...]

    rows = i * BLKA + lax.broadcasted_iota(jnp.int32, (BLKA, 1), 0)
    lpos = lax.rem(rows, L_SEQ)
    bidx = rows // L_SEQ
    idx_f = jnp.clip(lpos.astype(jnp.float32) + off, 0.0, float(L_SEQ - 1))
    i0f = jnp.floor(idx_f)
    w1_o[...] = idx_f - i0f
    i0 = i0f.astype(jnp.int32)
    i1 = jnp.minimum(i0 + 1, L_SEQ - 1)
    hcol = lax.broadcasted_iota(jnp.int32, (1, HP), 1) // P
    bhoff = (bidx * H + hcol) * L_SEQ
    g0_o[...] = bhoff + i0
    g1_o[...] = bhoff + i1


def _proj_call(xq, xkv, Wq, bq, Wk, bk, Wv, bv, Woff, boff, Waw, baw):
    n = xq.shape[0]
    grid = (n // BLKA,)
    row_spec = pl.BlockSpec((BLKA, DM), lambda i: (i, 0))
    hp_spec = pl.BlockSpec((BLKA, HP), lambda i: (i, 0))
    full = lambda shape: pl.BlockSpec(shape, lambda i: tuple(0 for _ in shape))
    return pl.pallas_call(
        _proj_body,
        grid=grid,
        in_specs=[
            row_spec, row_spec,
            full((DM, DM)), full((1, DM)),
            full((DM, DM)), full((1, DM)),
            full((DM, DM)), full((1, DM)),
            full((HP, DM)), full((1, HP)),
            full((HP, DM)), full((1, HP)),
        ],
        out_specs=[row_spec, row_spec, row_spec,
                   hp_spec, hp_spec, hp_spec, hp_spec],
        out_shape=[
            jax.ShapeDtypeStruct((n, DM), jnp.float32),
            jax.ShapeDtypeStruct((n, DM), jnp.float32),
            jax.ShapeDtypeStruct((n, DM), jnp.float32),
            jax.ShapeDtypeStruct((n, HP), jnp.int32),
            jax.ShapeDtypeStruct((n, HP), jnp.int32),
            jax.ShapeDtypeStruct((n, HP), jnp.float32),
            jax.ShapeDtypeStruct((n, HP), jnp.float32),
        ],
    )(xq, xkv, Wq, bq.reshape(1, DM), Wk, bk.reshape(1, DM),
      Wv, bv.reshape(1, DM), Woff, boff.reshape(1, HP),
      Waw, baw.reshape(1, HP))


def _outproj_body(x_ref, w_ref, b_ref, o_ref):
    o_ref[...] = lax.dot_general(
        x_ref[...], w_ref[...], (((1,), (1,)), ((), ())),
        preferred_element_type=jnp.float32) + b_ref[...]


def _outproj_call(x, Wout, bout):
    n = x.shape[0]
    row_spec = pl.BlockSpec((BLKA, DM), lambda i: (i, 0))
    return pl.pallas_call(
        _outproj_body,
        grid=(n // BLKA,),
        in_specs=[row_spec,
                  pl.BlockSpec((DM, DM), lambda i: (0, 0)),
                  pl.BlockSpec((1, DM), lambda i: (0, 0))],
        out_specs=row_spec,
        out_shape=jax.ShapeDtypeStruct((n, DM), jnp.float32),
    )(x, Wout, bout.reshape(1, DM))


def _sc_attend_body(kv_hbm, q_hbm, idx_hbm, w1_hbm, lg_hbm, out_hbm,
                    idx_v, kv_v, q_v, w1_v, lg_v, out_v, sem):
    cid = lax.axis_index("c")
    sid = lax.axis_index("s")
    bh = sid * _NC + cid
    b = bh // H
    h = lax.rem(bh, H)

    def chunk_body(c, carry):
        l0 = c * CH
        pltpu.sync_copy(idx_hbm.at[bh, pl.ds(c * ROWS, ROWS)], idx_v)
        cps = [pltpu.async_copy(kv_hbm.at[idx_v.at[pl.ds(j * 128, 128)]],
                                kv_v.at[pl.ds(j * 128, 128)], sem)
               for j in range(ROWS // 128)]
        pltpu.sync_copy(q_hbm.at[b, pl.ds(l0, CH), h], q_v)
        pltpu.sync_copy(w1_hbm.at[bh, pl.ds(c * CH * P, CH * P)], w1_v)
        pltpu.sync_copy(lg_hbm.at[bh, pl.ds(c * CH * P, CH * P)], lg_v)
        for cp in cps:
            cp.wait()

        def q_body(g, carry2):
            wvec = w1_v[pl.ds(g * 16, 16)]
            lvec = lg_v[pl.ds(g * 16, 16)]
            for qq in range(4):
                i = g * 4 + qq
                qi = [q_v[i, pl.ds(16 * t, 16)] for t in range(4)]
                base = i * P
                w1ps = []
                scrs = []
                for p in range(P):
                    w1p = jnp.full((16,), wvec[4 * qq + p])
                    lgp = lvec[4 * qq + p]
                    w1ps.append(w1p)
                    r0 = 2 * (base + p)
                    acc = jnp.zeros((16,), jnp.float32)
                    for t in range(4):
                        k0 = kv_v[r0, pl.ds(16 * t, 16)]
                        k1 = kv_v[r0 + 1, pl.ds(16 * t, 16)]
                        ki = k0 + w1p * (k1 - k0)
                        acc = acc + qi[t] * ki
                    scrs.append(lgp + jnp.sum(acc))
                m = jnp.maximum(jnp.maximum(scrs[0], scrs[1]),
                                jnp.maximum(scrs[2], scrs[3]))
                es = [jnp.exp(jnp.full((16,), s - m)) for s in scrs]
                den = (es[0] + es[1]) + (es[2] + es[3])
                wgt = [e / den for e in es]
                outt = [jnp.zeros((16,), jnp.float32) for _ in range(4)]
                for p in range(P):
                    r0 = 2 * (base + p)
                    for t in range(4):
                        v0 = kv_v[r0, pl.ds(64 + 16 * t, 16)]
                        v1 = kv_v[r0 + 1, pl.ds(64 + 16 * t, 16)]
                        vi = v0 + w1ps[p] * (v1 - v0)
                        outt[t] = outt[t] + wgt[p] * vi
                for t in range(4):
                    out_v[i, pl.ds(16 * t, 16)] = outt[t]
            return carry2

        lax.fori_loop(0, CH // 4, q_body, 0)
        pltpu.sync_copy(out_v, out_hbm.at[b, pl.ds(l0, CH), h])
        return carry

    lax.fori_loop(0, NCHUNK, chunk_body, 0)


@functools.lru_cache(maxsize=1)
def _build_sc_attend():
    return pl.kernel(
        _sc_attend_body,
        mesh=plsc.VectorSubcoreMesh(core_axis_name="c", subcore_axis_name="s"),
        compiler_params=pltpu.CompilerParams(needs_layout_passes=False),
        out_type=jax.ShapeDtypeStruct((B_SZ, L_SEQ, H, D), jnp.float32),
        scratch_types=[
            pltpu.VMEM((ROWS,), jnp.int32),
            pltpu.VMEM((ROWS, 2 * D), jnp.float32),
            pltpu.VMEM((CH, D), jnp.float32),
            pltpu.VMEM((CH * P,), jnp.float32),
            pltpu.VMEM((CH * P,), jnp.float32),
            pltpu.VMEM((CH, D), jnp.float32),
            pltpu.SemaphoreType.DMA,
        ],
    )


def kernel(q_in, kv_in, Wq, bq, Wk, bk, Wv, bv, Woff, boff, Waw, baw,
           Wout, bout):
    B, L, dm = q_in.shape
    xq = q_in.reshape(B * L, dm)
    xkv = kv_in.reshape(B * L, dm)
    q2, k2, v2, g0, g1, w1, lg = _proj_call(
        xq, xkv, Wq, bq, Wk, bk, Wv, bv, Woff, boff, Waw, baw)

    q4 = q2.reshape(B, L, H, D)
    kv = jnp.concatenate(
        [k2.reshape(B, L, H, D), v2.reshape(B, L, H, D)], axis=-1)
    kv_flat = kv.transpose(0, 2, 1, 3).reshape(B * H * L, 2 * D)
    idxp = (jnp.stack([g0.reshape(B, L, H, P), g1.reshape(B, L, H, P)],
                      axis=-1)
            .transpose(0, 2, 1, 3, 4).reshape(B * H, L * P * 2))
    w1s = w1.reshape(B, L, H, P).transpose(0, 2, 1, 3).reshape(B * H, L * P)
    lgs = lg.reshape(B, L, H, P).transpose(0, 2, 1, 3).reshape(B * H, L * P)

    ctx = _build_sc_attend()(kv_flat, q4, idxp, w1s, lgs)
    out = _outproj_call(ctx.reshape(B * L, dm), Wout, bout)
    return out.reshape(B, L, dm)
